# baseline (speedup 1.0000x reference)
import jax
import jax.numpy as jnp
from jax.experimental import pallas as pl
from jax.experimental.pallas import tpu as pltpu

D_MODEL = 4096
NUM_EXPERTS = 64
BLOCK_M = 256


def _router_kernel(idx_ref, x_ref, w_hbm, ot_ref, wbuf, wsem):
    @pl.when(pl.program_id(0) == 0)
    def _fetch_w():
        cp = pltpu.make_async_copy(w_hbm.at[idx_ref[0]], wbuf, wsem)
        cp.start()
        cp.wait()

    r = jax.lax.dot_general(
        x_ref[...],
        wbuf[...],
        (((1,), (1,)), ((), ())),
        preferred_element_type=jnp.float32,
    )
    ot_ref[...] = r.T


def kernel(index, hidden_states, W):
    m = hidden_states.shape[0]
    idx = jnp.asarray(index, dtype=jnp.int32).reshape((1,))
    grid_spec = pltpu.PrefetchScalarGridSpec(
        num_scalar_prefetch=1,
        grid=(m // BLOCK_M,),
        in_specs=[
            pl.BlockSpec((BLOCK_M, D_MODEL), lambda i, idx_ref: (i, 0)),
            pl.BlockSpec(memory_space=pl.ANY),
        ],
        out_specs=pl.BlockSpec((NUM_EXPERTS, BLOCK_M), lambda i, idx_ref: (0, i)),
        scratch_shapes=[
            pltpu.VMEM((NUM_EXPERTS, D_MODEL), jnp.float32),
            pltpu.SemaphoreType.DMA,
        ],
    )
    out_t = pl.pallas_call(
        _router_kernel,
        grid_spec=grid_spec,
        out_shape=jax.ShapeDtypeStruct((NUM_EXPERTS, m), jnp.float32),
    )(idx, hidden_states, W)
    return out_t.T


# speedup vs baseline: 1.1600x; 1.1600x over previous
import jax
import jax.numpy as jnp
from jax.experimental import pallas as pl
from jax.experimental.pallas import tpu as pltpu

D_MODEL = 4096
NUM_EXPERTS = 64
BLOCK_M = 512


def _router_kernel(idx_ref, x0_ref, x1_ref, w_hbm, ot_ref, wbuf, wsem):
    @pl.when(pl.program_id(0) == 0)
    def _fetch_w():
        cp = pltpu.make_async_copy(w_hbm.at[idx_ref[0]], wbuf, wsem)
        cp.start()
        cp.wait()

    dims = (((1,), (1,)), ((), ()))
    r0 = jax.lax.dot_general(
        x0_ref[...], wbuf[...], dims, preferred_element_type=jnp.float32
    )
    r1 = jax.lax.dot_general(
        x1_ref[...], wbuf[...], dims, preferred_element_type=jnp.float32
    )
    ot_ref[:, :BLOCK_M] = r0.T
    ot_ref[:, BLOCK_M:] = r1.T


def kernel(index, hidden_states, W):
    m = hidden_states.shape[0]
    idx = jnp.asarray(index, dtype=jnp.int32).reshape((1,))
    grid_spec = pltpu.PrefetchScalarGridSpec(
        num_scalar_prefetch=1,
        grid=(m // (2 * BLOCK_M),),
        in_specs=[
            pl.BlockSpec((BLOCK_M, D_MODEL), lambda i, idx_ref: (2 * i, 0)),
            pl.BlockSpec((BLOCK_M, D_MODEL), lambda i, idx_ref: (2 * i + 1, 0)),
            pl.BlockSpec(memory_space=pl.ANY),
        ],
        out_specs=pl.BlockSpec(
            (NUM_EXPERTS, 2 * BLOCK_M), lambda i, idx_ref: (0, i)
        ),
        scratch_shapes=[
            pltpu.VMEM((NUM_EXPERTS, D_MODEL), jnp.float32),
            pltpu.SemaphoreType.DMA,
        ],
    )
    out_t = pl.pallas_call(
        _router_kernel,
        grid_spec=grid_spec,
        out_shape=jax.ShapeDtypeStruct((NUM_EXPERTS, m), jnp.float32),
    )(idx, hidden_states, hidden_states, W)
    return out_t.T


# speedup vs baseline: 1.1736x; 1.0117x over previous
import jax
import jax.numpy as jnp
from jax.experimental import pallas as pl
from jax.experimental.pallas import tpu as pltpu

D_MODEL = 4096
NUM_EXPERTS = 64
BLOCK_M = 512


def _router_kernel(idx_ref, x_ref, w_hbm, ot_ref, wbuf, wsem):
    @pl.when(pl.program_id(0) == 0)
    def _fetch_w():
        cp = pltpu.make_async_copy(w_hbm.at[idx_ref[0]], wbuf, wsem)
        cp.start()
        cp.wait()

    r = jax.lax.dot_general(
        x_ref[...],
        wbuf[...],
        (((1,), (1,)), ((), ())),
        preferred_element_type=jnp.float32,
    )
    ot_ref[...] = r.T


def kernel(index, hidden_states, W):
    m = hidden_states.shape[0]
    idx = jnp.asarray(index, dtype=jnp.int32).reshape((1,))
    grid_spec = pltpu.PrefetchScalarGridSpec(
        num_scalar_prefetch=1,
        grid=(m // BLOCK_M,),
        in_specs=[
            pl.BlockSpec((BLOCK_M, D_MODEL), lambda i, idx_ref: (i, 0)),
            pl.BlockSpec(memory_space=pl.ANY),
        ],
        out_specs=pl.BlockSpec((NUM_EXPERTS, BLOCK_M), lambda i, idx_ref: (0, i)),
        scratch_shapes=[
            pltpu.VMEM((NUM_EXPERTS, D_MODEL), jnp.float32),
            pltpu.SemaphoreType.DMA,
        ],
    )
    out_t = pl.pallas_call(
        _router_kernel,
        grid_spec=grid_spec,
        out_shape=jax.ShapeDtypeStruct((NUM_EXPERTS, m), jnp.float32),
    )(idx, hidden_states, W)
    return out_t.T


# speedup vs baseline: 1.1745x; 1.0008x over previous
import jax
import jax.numpy as jnp
from jax.experimental import pallas as pl
from jax.experimental.pallas import tpu as pltpu

D_MODEL = 4096
NUM_EXPERTS = 64
BLOCK_M = 512


def _router_kernel(idx_ref, x_ref, w_hbm, ot_ref, wbuf, wsem):
    @pl.when(pl.program_id(0) == 0)
    def _fetch_w():
        cp = pltpu.make_async_copy(w_hbm.at[idx_ref[0]], wbuf, wsem)
        cp.start()
        cp.wait()

    r = jax.lax.dot_general(
        x_ref[...],
        wbuf[...],
        (((1,), (1,)), ((), ())),
        preferred_element_type=jnp.float32,
    )
    ot_ref[...] = r.T


def kernel(index, hidden_states, W):
    m = hidden_states.shape[0]
    idx = jnp.asarray(index, dtype=jnp.int32).reshape((1,))
    grid_spec = pltpu.PrefetchScalarGridSpec(
        num_scalar_prefetch=1,
        grid=(m // BLOCK_M,),
        in_specs=[
            pl.BlockSpec((BLOCK_M, D_MODEL), lambda i, idx_ref: (i, 0)),
            pl.BlockSpec(memory_space=pl.ANY),
        ],
        out_specs=pl.BlockSpec((NUM_EXPERTS, BLOCK_M), lambda i, idx_ref: (0, i)),
        scratch_shapes=[
            pltpu.VMEM((NUM_EXPERTS, D_MODEL), jnp.float32),
            pltpu.SemaphoreType.DMA,
        ],
    )
    out_t = pl.pallas_call(
        _router_kernel,
        grid_spec=grid_spec,
        out_shape=jax.ShapeDtypeStruct((NUM_EXPERTS, m), jnp.float32),
        compiler_params=pltpu.CompilerParams(
            dimension_semantics=("parallel",),
        ),
    )(idx, hidden_states, W)
    return out_t.T
